# baseline (speedup 1.0000x reference)
import jax
import jax.numpy as jnp
from jax import lax
from jax.experimental import pallas as pl
from jax.experimental.pallas import tpu as pltpu
from jax.experimental.pallas import tpu_sc as plsc

N_NODES = 10000
N_EDGES = 160000
D = 256
H = 128
L = 16
NC = 2
NS = 16
CHUNK = 80
NCHUNK = N_EDGES // CHUNK
TRIPS = NCHUNK // NS
ZCH = 80
NZCH = N_NODES // ZCH
ZTRIPS = (NZCH + NS - 1) // NS
GARBAGE = N_NODES
ACC_ROWS = N_NODES + 8


def _sc_body(xr_hbm, src_hbm, dst_hbm, out0_hbm, out1_hbm, cnt0_hbm, cnt1_hbm,
             sraw0, sraw1, draw0, draw1, gidx0, gidx1, didx0, didx1,
             rows0, rows1, ones_v, zc_v, acc, cntacc,
             sem_i0, sem_i1, sem_g0, sem_g1, sem_s0, sem_s1, sem_c0, sem_c1):
    c = lax.axis_index("c")
    s = lax.axis_index("s")

    sraw = (sraw0, sraw1)
    draw = (draw0, draw1)
    gidx = (gidx0, gidx1)
    didx = (didx0, didx1)
    rows = (rows0, rows1)
    sem_i = (sem_i0, sem_i1)
    sem_g = (sem_g0, sem_g1)
    sem_s = (sem_s0, sem_s1)
    sem_c = (sem_c0, sem_c1)

    zero16 = jnp.zeros((L,), jnp.float32)
    one16 = jnp.ones((L,), jnp.float32)

    @pl.loop(0, CHUNK)
    def _zr(r):
        for j in range(H // L):
            rows0[r, pl.ds(j * L, L)] = zero16
        ones_v[r, :] = one16
        zc_v[r, :] = zero16

    @pl.loop(0, ZTRIPS)
    def _za(i):
        ch = i * NS + s

        @pl.when(ch < NZCH)
        def _():
            r0 = ch * ZCH
            pltpu.sync_copy(rows0.at[0:ZCH], acc.at[pl.ds(r0, ZCH)])
            pltpu.sync_copy(zc_v, cntacc.at[pl.ds(r0, ZCH)])

    plsc.subcore_barrier()

    def _chunk_row(i):
        return s * TRIPS + jnp.minimum(i, TRIPS - 1)

    def _start_idx(b, i):
        r = _chunk_row(i)
        pltpu.async_copy(src_hbm.at[r], sraw[b], sem_i[b])
        pltpu.async_copy(dst_hbm.at[r], draw[b], sem_i[b])

    def _wait_idx(b, i):
        r = _chunk_row(i)
        pltpu.make_async_copy(src_hbm.at[r], sraw[b], sem_i[b]).wait()
        pltpu.make_async_copy(dst_hbm.at[r], draw[b], sem_i[b]).wait()

    def _compute(b):
        for j in range(CHUNK // L):
            sv = sraw[b][pl.ds(j * L, L)]
            dv = draw[b][pl.ds(j * L, L)]
            gidx[b][pl.ds(j * L, L)] = sv * 2 + c
            didx[b][pl.ds(j * L, L)] = jnp.where(
                sv == dv, jnp.int32(GARBAGE), dv)

    def _start_gather(b):
        pltpu.async_copy(xr_hbm.at[gidx[b]], rows[b], sem_g[b])

    def _wait_gather(b):
        pltpu.make_async_copy(xr_hbm.at[gidx[b]], rows[b], sem_g[b]).wait()

    def _counts_here(i):
        return ((s * TRIPS + i) & 1) == c

    def _start_scatter(b, i):
        pltpu.async_copy(rows[b], acc.at[didx[b]], sem_s[b], add=True)

        @pl.when(_counts_here(i))
        def _():
            pltpu.async_copy(ones_v, cntacc.at[didx[b]], sem_c[b], add=True)

    def _wait_scatter(b, i):
        pltpu.make_async_copy(rows[b], acc.at[didx[b]], sem_s[b]).wait()

        @pl.when(_counts_here(i))
        def _():
            pltpu.make_async_copy(ones_v, cntacc.at[didx[b]], sem_c[b]).wait()

    _start_idx(0, 0)
    _start_idx(1, 1)
    _wait_idx(0, 0)
    _compute(0)
    _start_gather(0)
    _start_idx(0, 2)
    _wait_idx(1, 1)
    _compute(1)
    _start_gather(1)
    _start_idx(1, 3)
    _wait_gather(0)
    _start_scatter(0, 0)

    @pl.loop(0, (TRIPS - 3) // 2)
    def _edges(j):
        i0 = 2 * j + 2
        _wait_scatter(0, i0 - 2)
        _wait_idx(0, i0)
        _compute(0)
        _start_gather(0)
        _start_idx(0, i0 + 2)
        _wait_gather(1)
        _start_scatter(1, i0 - 1)
        _wait_scatter(1, i0 - 1)
        _wait_idx(1, i0 + 1)
        _compute(1)
        _start_gather(1)
        _start_idx(1, i0 + 3)
        _wait_gather(0)
        _start_scatter(0, i0)

    _wait_scatter(0, TRIPS - 3)
    _wait_idx(0, TRIPS - 1)
    _compute(0)
    _start_gather(0)
    _wait_gather(1)
    _start_scatter(1, TRIPS - 2)
    _wait_gather(0)
    _start_scatter(0, TRIPS - 1)
    _wait_scatter(1, TRIPS - 2)
    _wait_scatter(0, TRIPS - 1)
    _wait_idx(1, TRIPS - 1)

    plsc.subcore_barrier()

    @pl.loop(0, ZTRIPS)
    def _out(i):
        ch = i * NS + s

        @pl.when(ch < NZCH)
        def _():
            r0 = ch * ZCH

            @pl.when(c == 0)
            def _():
                pltpu.sync_copy(acc.at[pl.ds(r0, ZCH)],
                                out0_hbm.at[pl.ds(r0, ZCH)])
                pltpu.sync_copy(cntacc.at[pl.ds(r0, ZCH)],
                                cnt0_hbm.at[pl.ds(r0, ZCH)])

            @pl.when(c == 1)
            def _():
                pltpu.sync_copy(acc.at[pl.ds(r0, ZCH)],
                                out1_hbm.at[pl.ds(r0, ZCH)])
                pltpu.sync_copy(cntacc.at[pl.ds(r0, ZCH)],
                                cnt1_hbm.at[pl.ds(r0, ZCH)])


def _make_sc_aggregate():
    mesh = plsc.VectorSubcoreMesh(core_axis_name="c", subcore_axis_name="s",
                                  num_cores=NC, num_subcores=NS)
    return pl.kernel(
        _sc_body,
        compiler_params=pltpu.CompilerParams(use_tc_tiling_on_sc=False),
        out_type=(
            jax.ShapeDtypeStruct((N_NODES, H), jnp.float32),
            jax.ShapeDtypeStruct((N_NODES, H), jnp.float32),
            jax.ShapeDtypeStruct((N_NODES, L), jnp.float32),
            jax.ShapeDtypeStruct((N_NODES, L), jnp.float32),
        ),
        mesh=mesh,
        scratch_types=(
            pltpu.VMEM((CHUNK,), jnp.int32),
            pltpu.VMEM((CHUNK,), jnp.int32),
            pltpu.VMEM((CHUNK,), jnp.int32),
            pltpu.VMEM((CHUNK,), jnp.int32),
            pltpu.VMEM((CHUNK,), jnp.int32),
            pltpu.VMEM((CHUNK,), jnp.int32),
            pltpu.VMEM((CHUNK,), jnp.int32),
            pltpu.VMEM((CHUNK,), jnp.int32),
            pltpu.VMEM((CHUNK, H), jnp.float32),
            pltpu.VMEM((CHUNK, H), jnp.float32),
            pltpu.VMEM((CHUNK, L), jnp.float32),
            pltpu.VMEM((ZCH, L), jnp.float32),
            pltpu.VMEM_SHARED((ACC_ROWS, H), jnp.float32),
            pltpu.VMEM_SHARED((ACC_ROWS, L), jnp.float32),
            pltpu.SemaphoreType.DMA,
            pltpu.SemaphoreType.DMA,
            pltpu.SemaphoreType.DMA,
            pltpu.SemaphoreType.DMA,
            pltpu.SemaphoreType.DMA,
            pltpu.SemaphoreType.DMA,
            pltpu.SemaphoreType.DMA,
            pltpu.SemaphoreType.DMA,
        ),
    )


def _mlp_body(x_ref, a0_ref, a1_ref, c0_ref, c1_ref, w1_ref, b1_ref, w2_ref,
              b2_ref, o_ref):
    cnt = c0_ref[:, 0:1] + c1_ref[:, 0:1]
    inv = 1.0 / jnp.maximum(cnt, 1.0)
    m = jnp.concatenate([a0_ref[...], a1_ref[...]], axis=1) * inv
    h = x_ref[...] + m
    h = jnp.dot(h, w1_ref[...], preferred_element_type=jnp.float32) + b1_ref[...]
    h = jnp.where(h >= 0, h, 0.01 * h)
    h = jnp.dot(h, w2_ref[...], preferred_element_type=jnp.float32) + b2_ref[...]
    o_ref[...] = jnp.where(h >= 0, h, 0.01 * h)


_BLK = 1000


def _mlp(x, a0, a1, cnt0, cnt1, W1, b1, W2, b2):
    grid = (N_NODES // _BLK,)
    return pl.pallas_call(
        _mlp_body,
        grid=grid,
        in_specs=[
            pl.BlockSpec((_BLK, D), lambda i: (i, 0)),
            pl.BlockSpec((_BLK, H), lambda i: (i, 0)),
            pl.BlockSpec((_BLK, H), lambda i: (i, 0)),
            pl.BlockSpec((_BLK, L), lambda i: (i, 0)),
            pl.BlockSpec((_BLK, L), lambda i: (i, 0)),
            pl.BlockSpec((D, D), lambda i: (0, 0)),
            pl.BlockSpec((1, D), lambda i: (0, 0)),
            pl.BlockSpec((D, D), lambda i: (0, 0)),
            pl.BlockSpec((1, D), lambda i: (0, 0)),
        ],
        out_specs=pl.BlockSpec((_BLK, D), lambda i: (i, 0)),
        out_shape=jax.ShapeDtypeStruct((N_NODES, D), jnp.float32),
    )(x, a0, a1, cnt0, cnt1, W1, b1, W2, b2)


def kernel(x, edge_index, W1, b1, W2, b2):
    src = edge_index[0].reshape(NCHUNK, CHUNK)
    dst = edge_index[1].reshape(NCHUNK, CHUNK)
    xr = x.reshape(2 * N_NODES, H)
    sc = _make_sc_aggregate()
    summed0, summed1, cnt0, cnt1 = sc(xr, src, dst)
    return _mlp(x, summed0, summed1, cnt0, cnt1, W1, b1.reshape(1, D),
                W2, b2.reshape(1, D))


# speedup vs baseline: 1.1366x; 1.1366x over previous
import jax
import jax.numpy as jnp
from jax import lax
from jax.experimental import pallas as pl
from jax.experimental.pallas import tpu as pltpu
from jax.experimental.pallas import tpu_sc as plsc

N_NODES = 10000
N_EDGES = 160000
D = 256
H = 128
L = 16
NC = 2
NS = 16
CHUNK = 80
NCHUNK = N_EDGES // CHUNK
TRIPS = NCHUNK // NS
ZCH = 80
NZCH = N_NODES // ZCH
ZTRIPS = (NZCH + NS - 1) // NS
GARBAGE = N_NODES
ACC_ROWS = N_NODES + 8


def _sc_body(xr_hbm, src_hbm, dst_hbm, out0_hbm, out1_hbm, cnt0_hbm, cnt1_hbm,
             sraw0, sraw1, sraw2, draw0, draw1, draw2,
             gidx0, gidx1, gidx2, didx0, didx1, didx2,
             rows0, rows1, rows2, ones_v, zc_v, acc, cntacc,
             sem_i0, sem_i1, sem_i2, sem_g0, sem_g1, sem_g2,
             sem_s0, sem_s1, sem_s2, sem_c0, sem_c1, sem_c2):
    c = lax.axis_index("c")
    s = lax.axis_index("s")

    sraw = (sraw0, sraw1, sraw2)
    draw = (draw0, draw1, draw2)
    gidx = (gidx0, gidx1, gidx2)
    didx = (didx0, didx1, didx2)
    rows = (rows0, rows1, rows2)
    sem_i = (sem_i0, sem_i1, sem_i2)
    sem_g = (sem_g0, sem_g1, sem_g2)
    sem_s = (sem_s0, sem_s1, sem_s2)
    sem_c = (sem_c0, sem_c1, sem_c2)

    zero16 = jnp.zeros((L,), jnp.float32)
    one16 = jnp.ones((L,), jnp.float32)

    @pl.loop(0, CHUNK)
    def _zr(r):
        for j in range(H // L):
            rows0[r, pl.ds(j * L, L)] = zero16
        ones_v[r, :] = one16
        zc_v[r, :] = zero16

    @pl.loop(0, ZTRIPS)
    def _za(i):
        ch = i * NS + s

        @pl.when(ch < NZCH)
        def _():
            r0 = ch * ZCH
            pltpu.sync_copy(rows0.at[0:ZCH], acc.at[pl.ds(r0, ZCH)])
            pltpu.sync_copy(zc_v, cntacc.at[pl.ds(r0, ZCH)])

    plsc.subcore_barrier()

    def _chunk_row(i):
        return s * TRIPS + jnp.minimum(i, TRIPS - 1)

    def _start_idx(b, i):
        r = _chunk_row(i)
        pltpu.async_copy(src_hbm.at[r], sraw[b], sem_i[b])
        pltpu.async_copy(dst_hbm.at[r], draw[b], sem_i[b])

    def _wait_idx(b, i):
        r = _chunk_row(i)
        pltpu.make_async_copy(src_hbm.at[r], sraw[b], sem_i[b]).wait()
        pltpu.make_async_copy(dst_hbm.at[r], draw[b], sem_i[b]).wait()

    def _compute(b):
        for j in range(CHUNK // L):
            sv = sraw[b][pl.ds(j * L, L)]
            dv = draw[b][pl.ds(j * L, L)]
            gidx[b][pl.ds(j * L, L)] = sv * 2 + c
            didx[b][pl.ds(j * L, L)] = jnp.where(
                sv == dv, jnp.int32(GARBAGE), dv)

    def _start_gather(b):
        pltpu.async_copy(xr_hbm.at[gidx[b]], rows[b], sem_g[b])

    def _wait_gather(b):
        pltpu.make_async_copy(xr_hbm.at[gidx[b]], rows[b], sem_g[b]).wait()

    def _counts_here(i):
        return ((s * TRIPS + i) & 1) == c

    def _start_scatter(b, i):
        pltpu.async_copy(rows[b], acc.at[didx[b]], sem_s[b], add=True)

        @pl.when(_counts_here(i))
        def _():
            pltpu.async_copy(ones_v, cntacc.at[didx[b]], sem_c[b], add=True)

    def _wait_scatter(b, i):
        pltpu.make_async_copy(rows[b], acc.at[didx[b]], sem_s[b]).wait()

        @pl.when(_counts_here(i))
        def _():
            pltpu.make_async_copy(ones_v, cntacc.at[didx[b]], sem_c[b]).wait()

    _start_idx(0, 0)
    _start_idx(1, 1)
    _start_idx(2, 2)
    _wait_idx(0, 0)
    _compute(0)
    _start_gather(0)
    _start_idx(0, 3)
    _wait_idx(1, 1)
    _compute(1)
    _start_gather(1)
    _start_idx(1, 4)
    _wait_gather(0)
    _start_scatter(0, 0)
    _wait_idx(2, 2)
    _compute(2)
    _start_gather(2)
    _start_idx(2, 5)
    _wait_gather(1)
    _start_scatter(1, 1)

    def _slot(b, i):
        bp = (b + 2) % 3
        _wait_scatter(b, i - 3)
        _wait_idx(b, i)
        _compute(b)
        _start_gather(b)
        _start_idx(b, i + 3)
        _wait_gather(bp)
        _start_scatter(bp, i - 1)

    @pl.loop(0, (TRIPS - 5) // 3)
    def _edges(j):
        i0 = 3 * j + 3
        _slot(0, i0)
        _slot(1, i0 + 1)
        _slot(2, i0 + 2)

    _wait_scatter(0, TRIPS - 5)
    _wait_idx(0, TRIPS - 2)
    _compute(0)
    _start_gather(0)
    _wait_gather(2)
    _start_scatter(2, TRIPS - 3)
    _wait_scatter(1, TRIPS - 4)
    _wait_idx(1, TRIPS - 1)
    _compute(1)
    _start_gather(1)
    _wait_gather(0)
    _start_scatter(0, TRIPS - 2)
    _wait_gather(1)
    _start_scatter(1, TRIPS - 1)
    _wait_scatter(2, TRIPS - 3)
    _wait_scatter(0, TRIPS - 2)
    _wait_scatter(1, TRIPS - 1)
    _wait_idx(2, TRIPS - 1)

    plsc.subcore_barrier()

    @pl.loop(0, ZTRIPS)
    def _out(i):
        ch = i * NS + s

        @pl.when(ch < NZCH)
        def _():
            r0 = ch * ZCH
            pltpu.sync_copy(acc.at[pl.ds(r0, ZCH)], rows0.at[0:ZCH])
            pltpu.sync_copy(cntacc.at[pl.ds(r0, ZCH)], zc_v)

            @pl.when(c == 0)
            def _():
                pltpu.sync_copy(rows0.at[0:ZCH], out0_hbm.at[pl.ds(r0, ZCH)])
                pltpu.sync_copy(zc_v, cnt0_hbm.at[pl.ds(r0, ZCH)])

            @pl.when(c == 1)
            def _():
                pltpu.sync_copy(rows0.at[0:ZCH], out1_hbm.at[pl.ds(r0, ZCH)])
                pltpu.sync_copy(zc_v, cnt1_hbm.at[pl.ds(r0, ZCH)])


def _make_sc_aggregate():
    mesh = plsc.VectorSubcoreMesh(core_axis_name="c", subcore_axis_name="s",
                                  num_cores=NC, num_subcores=NS)
    return pl.kernel(
        _sc_body,
        compiler_params=pltpu.CompilerParams(use_tc_tiling_on_sc=False),
        out_type=(
            jax.ShapeDtypeStruct((N_NODES, H), jnp.float32),
            jax.ShapeDtypeStruct((N_NODES, H), jnp.float32),
            jax.ShapeDtypeStruct((N_NODES, L), jnp.float32),
            jax.ShapeDtypeStruct((N_NODES, L), jnp.float32),
        ),
        mesh=mesh,
        scratch_types=(
            pltpu.VMEM((CHUNK,), jnp.int32),
            pltpu.VMEM((CHUNK,), jnp.int32),
            pltpu.VMEM((CHUNK,), jnp.int32),
            pltpu.VMEM((CHUNK,), jnp.int32),
            pltpu.VMEM((CHUNK,), jnp.int32),
            pltpu.VMEM((CHUNK,), jnp.int32),
            pltpu.VMEM((CHUNK,), jnp.int32),
            pltpu.VMEM((CHUNK,), jnp.int32),
            pltpu.VMEM((CHUNK,), jnp.int32),
            pltpu.VMEM((CHUNK,), jnp.int32),
            pltpu.VMEM((CHUNK,), jnp.int32),
            pltpu.VMEM((CHUNK,), jnp.int32),
            pltpu.VMEM((CHUNK, H), jnp.float32),
            pltpu.VMEM((CHUNK, H), jnp.float32),
            pltpu.VMEM((CHUNK, H), jnp.float32),
            pltpu.VMEM((CHUNK, L), jnp.float32),
            pltpu.VMEM((ZCH, L), jnp.float32),
            pltpu.VMEM_SHARED((ACC_ROWS, H), jnp.float32),
            pltpu.VMEM_SHARED((ACC_ROWS, L), jnp.float32),
            pltpu.SemaphoreType.DMA,
            pltpu.SemaphoreType.DMA,
            pltpu.SemaphoreType.DMA,
            pltpu.SemaphoreType.DMA,
            pltpu.SemaphoreType.DMA,
            pltpu.SemaphoreType.DMA,
            pltpu.SemaphoreType.DMA,
            pltpu.SemaphoreType.DMA,
            pltpu.SemaphoreType.DMA,
            pltpu.SemaphoreType.DMA,
            pltpu.SemaphoreType.DMA,
            pltpu.SemaphoreType.DMA,
        ),
    )


def _mlp_body(x_ref, a0_ref, a1_ref, c0_ref, c1_ref, w1_ref, b1_ref, w2_ref,
              b2_ref, o_ref):
    cnt = c0_ref[:, 0:1] + c1_ref[:, 0:1]
    inv = 1.0 / jnp.maximum(cnt, 1.0)
    m = jnp.concatenate([a0_ref[...], a1_ref[...]], axis=1) * inv
    h = x_ref[...] + m
    h = jnp.dot(h, w1_ref[...], preferred_element_type=jnp.float32) + b1_ref[...]
    h = jnp.where(h >= 0, h, 0.01 * h)
    h = jnp.dot(h, w2_ref[...], preferred_element_type=jnp.float32) + b2_ref[...]
    o_ref[...] = jnp.where(h >= 0, h, 0.01 * h)


_BLK = 1000


def _mlp(x, a0, a1, cnt0, cnt1, W1, b1, W2, b2):
    grid = (N_NODES // _BLK,)
    return pl.pallas_call(
        _mlp_body,
        grid=grid,
        in_specs=[
            pl.BlockSpec((_BLK, D), lambda i: (i, 0)),
            pl.BlockSpec((_BLK, H), lambda i: (i, 0)),
            pl.BlockSpec((_BLK, H), lambda i: (i, 0)),
            pl.BlockSpec((_BLK, L), lambda i: (i, 0)),
            pl.BlockSpec((_BLK, L), lambda i: (i, 0)),
            pl.BlockSpec((D, D), lambda i: (0, 0)),
            pl.BlockSpec((1, D), lambda i: (0, 0)),
            pl.BlockSpec((D, D), lambda i: (0, 0)),
            pl.BlockSpec((1, D), lambda i: (0, 0)),
        ],
        out_specs=pl.BlockSpec((_BLK, D), lambda i: (i, 0)),
        out_shape=jax.ShapeDtypeStruct((N_NODES, D), jnp.float32),
    )(x, a0, a1, cnt0, cnt1, W1, b1, W2, b2)


def kernel(x, edge_index, W1, b1, W2, b2):
    src = edge_index[0].reshape(NCHUNK, CHUNK)
    dst = edge_index[1].reshape(NCHUNK, CHUNK)
    xr = x.reshape(2 * N_NODES, H)
    sc = _make_sc_aggregate()
    summed0, summed1, cnt0, cnt1 = sc(xr, src, dst)
    return _mlp(x, summed0, summed1, cnt0, cnt1, W1, b1.reshape(1, D),
                W2, b2.reshape(1, D))
